# baseline (speedup 1.0000x reference)
import functools

import jax
import jax.numpy as jnp
from jax import lax
from jax.experimental import pallas as pl
from jax.experimental.pallas import tpu as pltpu
from jax.experimental.pallas import tpu_sc as plsc

D = 128
B = 16384
L = 16
NC, NS = 2, 16
NW = NC * NS
BPW = B // NW
CHUNK = 128
NCHUNK = BPW // CHUNK


def _build():
    mesh = plsc.VectorSubcoreMesh(core_axis_name="c", subcore_axis_name="s")

    @functools.partial(
        pl.kernel,
        mesh=mesh,
        out_type=jax.ShapeDtypeStruct((B, D), jnp.float32),
        scratch_types=(
            [pltpu.VMEM((NCHUNK, CHUNK), jnp.int32)]
            + [pltpu.VMEM((CHUNK, D), jnp.float32)] * NCHUNK
            + [pltpu.SemaphoreType.DMA] * (2 * NCHUNK)
        ),
    )
    def gather_sigmoid(table_hbm, idx_hbm, out_hbm, idx_v, *scr):
        bufs = scr[:NCHUNK]
        gsems = scr[NCHUNK:2 * NCHUNK]
        wsems = scr[2 * NCHUNK:]
        wid = lax.axis_index("s") * NC + lax.axis_index("c")
        base = wid * BPW
        pltpu.sync_copy(idx_hbm.at[wid], idx_v)

        gathers = [
            pltpu.async_copy(table_hbm.at[idx_v.at[j]], bufs[j], gsems[j])
            for j in range(NCHUNK)
        ]

        def sigmoid_inplace(buf):
            @plsc.parallel_loop(0, CHUNK, step=1, unroll=8)
            def _rows(r):
                for k in range(D // L):
                    sl = pl.ds(k * L, L)
                    x = buf[r, sl]
                    buf[r, sl] = 1.0 / (1.0 + jnp.exp(-x))

        writes = []
        for j in range(NCHUNK):
            gathers[j].wait()
            sigmoid_inplace(bufs[j])
            writes.append(pltpu.async_copy(
                bufs[j], out_hbm.at[pl.ds(base + j * CHUNK, CHUNK)],
                wsems[j]))
        for w in writes:
            w.wait()

    return gather_sigmoid


_GATHER_SIGMOID = _build()


def kernel(slice_num, optimized_array):
    idx = slice_num.reshape(NW, NCHUNK, CHUNK)
    return _GATHER_SIGMOID(optimized_array, idx)


# speedup vs baseline: 1.0528x; 1.0528x over previous
import functools

import jax
import jax.numpy as jnp
from jax import lax
from jax.experimental import pallas as pl
from jax.experimental.pallas import tpu as pltpu
from jax.experimental.pallas import tpu_sc as plsc

D = 128
B = 16384
L = 16
NC, NS = 2, 16
NW = NC * NS
BPW = B // NW
CHUNK = 128
NCHUNK = BPW // CHUNK


def _build():
    mesh = plsc.VectorSubcoreMesh(core_axis_name="c", subcore_axis_name="s")

    @functools.partial(
        pl.kernel,
        mesh=mesh,
        out_type=jax.ShapeDtypeStruct((B, D), jnp.float32),
        scratch_types=(
            [pltpu.VMEM((NCHUNK, CHUNK), jnp.int32)]
            + [pltpu.VMEM((CHUNK, D), jnp.float32)] * NCHUNK
            + [pltpu.SemaphoreType.DMA] * (2 * NCHUNK)
        ),
    )
    def gather_sigmoid(table_hbm, idx_hbm, out_hbm, idx_v, *scr):
        bufs = scr[:NCHUNK]
        gsems = scr[NCHUNK:2 * NCHUNK]
        wsems = scr[2 * NCHUNK:]
        wid = lax.axis_index("s") * NC + lax.axis_index("c")
        base = wid * BPW
        pltpu.sync_copy(idx_hbm.at[wid], idx_v)

        gathers = [
            pltpu.async_copy(table_hbm.at[idx_v.at[j]], bufs[j], gsems[j])
            for j in range(NCHUNK)
        ]

        def sigmoid_inplace(buf):
            @plsc.parallel_loop(0, CHUNK, step=1, unroll=2)
            def _rows(r):
                for k in range(D // L):
                    sl = pl.ds(k * L, L)
                    x = buf[r, sl]
                    buf[r, sl] = 1.0 / (1.0 + jnp.exp(-x))

        writes = []
        for j in range(NCHUNK):
            gathers[j].wait()
            sigmoid_inplace(bufs[j])
            writes.append(pltpu.async_copy(
                bufs[j], out_hbm.at[pl.ds(base + j * CHUNK, CHUNK)],
                wsems[j]))
        for w in writes:
            w.wait()

    return gather_sigmoid


_GATHER_SIGMOID = _build()


def kernel(slice_num, optimized_array):
    idx = slice_num.reshape(NW, NCHUNK, CHUNK)
    return _GATHER_SIGMOID(optimized_array, idx)
